# baseline (speedup 1.0000x reference)
import functools

import jax
import jax.numpy as jnp
import numpy as np
from jax import lax
from jax.experimental import pallas as pl
from jax.experimental.pallas import tpu as pltpu
from jax.experimental.pallas import tpu_sc as plsc

W_IMG = 512
H_IMG = 512
N_BOXES = 512
MNT = 63
MNT_STRIDE = 64
N_TILES = 32
ROWS_PER_TILE = W_IMG // N_TILES
L = 16
TDIM = 128
NF = 16

_A = np.arange(TDIM, dtype=np.int64)
_D = np.maximum(_A, 1)[:, None]
_CTABLE = np.where(_A[None, :] < _D, (_A[None, :] * MNT) // _D, MNT)
_CTABLE = _CTABLE.astype(np.int32).reshape(TDIM * TDIM)


def _render_body(boxes_hbm, mnt_hbm, ctab_hbm, out_hbm,
                 boxes_v, mnt_v, ctab_v, fields_v, canvas, sem):
    cid = lax.axis_index("c")
    sid = lax.axis_index("s")
    t = sid * 2 + cid

    pltpu.sync_copy(boxes_hbm, boxes_v)
    pltpu.sync_copy(mnt_hbm, mnt_v)
    pltpu.sync_copy(ctab_hbm, ctab_v)

    lanes = lax.iota(jnp.int32, L)
    lanes32 = lanes * N_TILES

    def coord_body(k, carry):
        fb = (k * L + lanes) * 4
        cx = plsc.load_gather(boxes_v, [fb])
        cy = plsc.load_gather(boxes_v, [fb + 1])
        bw = plsc.load_gather(boxes_v, [fb + 2])
        bh = plsc.load_gather(boxes_v, [fb + 3])
        x1 = ((cx - 0.5 * bw) * float(W_IMG)).astype(jnp.int32)
        y1 = ((cy - 0.5 * bh) * float(W_IMG)).astype(jnp.int32)
        x2 = ((cx + 0.5 * bw) * float(W_IMG)).astype(jnp.int32)
        y2 = ((cy + 0.5 * bh) * float(W_IMG)).astype(jnp.int32)
        w = x2 - x1
        h = y2 - y1
        wb = jnp.minimum(jnp.maximum(w, 1), TDIM - 1) << 7
        hb = jnp.minimum(jnp.maximum(h, 1), TDIM - 1) << 7
        dx0 = (t - x1) & (N_TILES - 1)
        rl0 = (x1 + dx0 - t) >> 5
        nc4 = (h + 4 * L - 1) >> 6
        off0 = (rl0 << 9) + y1
        off1 = (jnp.minimum(rl0 + 1, ROWS_PER_TILE) << 9) + y1
        off2 = (jnp.minimum(rl0 + 2, ROWS_PER_TILE) << 9) + y1
        off3 = (jnp.minimum(rl0 + 3, ROWS_PER_TILE) << 9) + y1
        fo = (k * L + lanes) * NF
        plsc.store_scatter(fields_v, [fo], wb + dx0)
        plsc.store_scatter(fields_v, [fo + 1], wb + TDIM - 1)
        plsc.store_scatter(fields_v, [fo + 2], hb)
        plsc.store_scatter(fields_v, [fo + 3], nc4)
        plsc.store_scatter(fields_v, [fo + 4], off0)
        plsc.store_scatter(fields_v, [fo + 5], off1)
        plsc.store_scatter(fields_v, [fo + 6], off2)
        plsc.store_scatter(fields_v, [fo + 7], off3)
        plsc.store_scatter(fields_v, [fo + 8], w - dx0)
        return carry

    lax.fori_loop(0, N_BOXES // L, coord_body, 0)

    def zero_body(k, carry):
        for u in range(4):
            canvas[pl.ds((k * 4 + u) * L, L)] = jnp.zeros((L,), jnp.float32)
        return carry

    lax.fori_loop(0, ((ROWS_PER_TILE + 1) * H_IMG) // (4 * L), zero_body, 0)

    def box_body(b, fvec):
        nvec = plsc.load_gather(fields_v, [(b + 1) * NF + lanes])
        wbdx = fvec[0]
        wb127 = fvec[1]
        hb = fvec[2]
        nc4 = fvec[3]
        off0 = fvec[4]
        off1 = fvec[5]
        off2 = fvec[6]
        off3 = fvec[7]
        nfl = fvec[8]
        ridx = jnp.minimum(wbdx + lanes32, wb127)
        rvec = plsc.load_gather(ctab_v, [ridx]) << 6
        rb0 = rvec[0]
        rb1 = rvec[1]

        def col01(jj, c2):
            base = hb + jj * (4 * L)
            ccs = [ctab_v[pl.ds(base + u * L, L)] for u in range(4)]
            for rb, off in ((rb0, off0), (rb1, off1)):
                for u in range(4):
                    v0 = plsc.load_gather(mnt_v, [rb + ccs[u]])
                    s0 = pl.ds(off + jj * (4 * L) + u * L, L)
                    canvas[s0] = jnp.maximum(canvas[s0], v0)
            return c2

        lax.fori_loop(0, nc4, col01, 0)

        @pl.when(nfl > 2 * N_TILES)
        def _rows23():
            rb2 = rvec[2]
            rb3 = rvec[3]

            def col23(jj, c2):
                base = hb + jj * (4 * L)
                ccs = [ctab_v[pl.ds(base + u * L, L)] for u in range(4)]
                for rb, off in ((rb2, off2), (rb3, off3)):
                    for u in range(4):
                        v0 = plsc.load_gather(mnt_v, [rb + ccs[u]])
                        s0 = pl.ds(off + jj * (4 * L) + u * L, L)
                        canvas[s0] = jnp.maximum(canvas[s0], v0)
                return c2

            lax.fori_loop(0, nc4, col23, 0)

        return nvec

    fvec0 = plsc.load_gather(fields_v, [lanes])
    lax.fori_loop(0, N_BOXES, box_body, fvec0)

    copies = []
    for k in range(ROWS_PER_TILE):
        dst_off = pl.multiple_of((t + N_TILES * k) * H_IMG, 512)
        copies.append(pltpu.async_copy(
            canvas.at[pl.ds(k * H_IMG, H_IMG)],
            out_hbm.at[pl.ds(dst_off, H_IMG)], sem))
    for c in copies:
        c.wait()


@jax.jit
def _render(boxes_flat, mnt_flat):
    mesh = plsc.VectorSubcoreMesh(core_axis_name="c", subcore_axis_name="s")
    f = functools.partial(
        pl.kernel,
        mesh=mesh,
        compiler_params=pltpu.CompilerParams(needs_layout_passes=False),
        out_type=jax.ShapeDtypeStruct((W_IMG * H_IMG,), jnp.float32),
        scratch_types=[
            pltpu.VMEM((N_BOXES * 4,), jnp.float32),
            pltpu.VMEM((MNT_STRIDE * MNT_STRIDE,), jnp.float32),
            pltpu.VMEM((TDIM * TDIM,), jnp.int32),
            pltpu.VMEM(((N_BOXES + 1) * NF + L,), jnp.int32),
            pltpu.VMEM(((ROWS_PER_TILE + 1) * H_IMG,), jnp.float32),
            pltpu.SemaphoreType.DMA,
        ],
    )(_render_body)
    return f(boxes_flat, mnt_flat, jnp.asarray(_CTABLE))


def kernel(boxes, mount):
    mnt2d = mount[0, 0]
    mnt_flat = jnp.pad(mnt2d, ((0, MNT_STRIDE - MNT), (0, MNT_STRIDE - MNT)))
    mnt_flat = mnt_flat.reshape(MNT_STRIDE * MNT_STRIDE)
    boxes_flat = boxes.reshape(N_BOXES * 4)
    out = _render(boxes_flat, mnt_flat)
    return out.reshape(1, 1, W_IMG, H_IMG)


# speedup vs baseline: 1.2237x; 1.2237x over previous
import functools

import jax
import jax.numpy as jnp
import numpy as np
from jax import lax
from jax.experimental import pallas as pl
from jax.experimental.pallas import tpu as pltpu
from jax.experimental.pallas import tpu_sc as plsc

W_IMG = 512
H_IMG = 512
N_BOXES = 512
MNT = 63
MNT_STRIDE = 64
N_TILES = 32
ROWS_PER_TILE = W_IMG // N_TILES
L = 16
TDIM = 128
NF = 16

_A = np.arange(TDIM, dtype=np.int64)
_D = np.maximum(_A, 1)[:, None]
_CTABLE = np.where(_A[None, :] < _D, (_A[None, :] * MNT) // _D, MNT)
_CTABLE = _CTABLE.astype(np.int32).reshape(TDIM * TDIM)


def _render_body(boxes_hbm, mnt_hbm, ctab_hbm, out_hbm,
                 boxes_v, mnt_v, ctab_v, fields_v, canvas, sem):
    cid = lax.axis_index("c")
    sid = lax.axis_index("s")
    t = sid * 2 + cid

    c1 = pltpu.async_copy(boxes_hbm, boxes_v, sem)
    c2 = pltpu.async_copy(mnt_hbm, mnt_v, sem)
    c3 = pltpu.async_copy(ctab_hbm, ctab_v, sem)

    lanes = lax.iota(jnp.int32, L)
    lanes32 = lanes * N_TILES

    def coord_body(k, carry):
        fb = (k * L + lanes) * 4
        cx = plsc.load_gather(boxes_v, [fb])
        cy = plsc.load_gather(boxes_v, [fb + 1])
        bw = plsc.load_gather(boxes_v, [fb + 2])
        bh = plsc.load_gather(boxes_v, [fb + 3])
        x1 = ((cx - 0.5 * bw) * float(W_IMG)).astype(jnp.int32)
        y1 = ((cy - 0.5 * bh) * float(W_IMG)).astype(jnp.int32)
        x2 = ((cx + 0.5 * bw) * float(W_IMG)).astype(jnp.int32)
        y2 = ((cy + 0.5 * bh) * float(W_IMG)).astype(jnp.int32)
        w = x2 - x1
        h = y2 - y1
        wb = jnp.minimum(jnp.maximum(w, 1), TDIM - 1) << 7
        hb = jnp.minimum(jnp.maximum(h, 1), TDIM - 1) << 7
        dx0 = (t - x1) & (N_TILES - 1)
        rl0 = (x1 + dx0 - t) >> 5
        nc2 = (h + 2 * L - 1) >> 5
        off0 = (rl0 << 9) + y1
        off1 = (jnp.minimum(rl0 + 1, ROWS_PER_TILE) << 9) + y1
        off2 = (jnp.minimum(rl0 + 2, ROWS_PER_TILE) << 9) + y1
        off3 = (jnp.minimum(rl0 + 3, ROWS_PER_TILE) << 9) + y1
        fo = (k * L + lanes) * NF
        plsc.store_scatter(fields_v, [fo], wb + dx0)
        plsc.store_scatter(fields_v, [fo + 1], wb + TDIM - 1)
        plsc.store_scatter(fields_v, [fo + 2], hb)
        plsc.store_scatter(fields_v, [fo + 3], nc2)
        plsc.store_scatter(fields_v, [fo + 4], off0)
        plsc.store_scatter(fields_v, [fo + 5], off1)
        plsc.store_scatter(fields_v, [fo + 6], off2)
        plsc.store_scatter(fields_v, [fo + 7], off3)
        plsc.store_scatter(fields_v, [fo + 8], w - dx0)
        return carry

    def zero_body(k, carry):
        for u in range(4):
            canvas[pl.ds((k * 4 + u) * L, L)] = jnp.zeros((L,), jnp.float32)
        return carry

    lax.fori_loop(0, ((ROWS_PER_TILE + 1) * H_IMG) // (4 * L), zero_body, 0)

    c1.wait()
    c2.wait()
    c3.wait()

    lax.fori_loop(0, N_BOXES // L, coord_body, 0)
    plsc.store_scatter(fields_v, [N_BOXES * NF + lanes],
                       jnp.zeros((L,), jnp.int32))

    def box_body(b, carry):
        fvec, rvec = carry
        nvec = plsc.load_gather(fields_v, [(b + 1) * NF + lanes])
        hb = fvec[2]
        nc2 = fvec[3]
        off0 = fvec[4]
        off1 = fvec[5]
        off2 = fvec[6]
        off3 = fvec[7]
        nfl = fvec[8]
        rb0 = rvec[0]
        rb1 = rvec[1]

        def col01(jj, c2):
            base = hb + jj * (2 * L)
            cc0 = ctab_v[pl.ds(base, L)]
            cc1 = ctab_v[pl.ds(base + L, L)]
            for rb, off in ((rb0, off0), (rb1, off1)):
                v0 = plsc.load_gather(mnt_v, [rb + cc0])
                v1 = plsc.load_gather(mnt_v, [rb + cc1])
                s0 = pl.ds(off + jj * (2 * L), L)
                s1 = pl.ds(off + jj * (2 * L) + L, L)
                canvas[s0] = jnp.maximum(canvas[s0], v0)
                canvas[s1] = jnp.maximum(canvas[s1], v1)
            return c2

        lax.fori_loop(0, nc2, col01, 0)

        @pl.when(nfl > 2 * N_TILES)
        def _rows23():
            rb2 = rvec[2]
            rb3 = rvec[3]

            def col23(jj, c2):
                base = hb + jj * (2 * L)
                cc0 = ctab_v[pl.ds(base, L)]
                cc1 = ctab_v[pl.ds(base + L, L)]
                for rb, off in ((rb2, off2), (rb3, off3)):
                    v0 = plsc.load_gather(mnt_v, [rb + cc0])
                    v1 = plsc.load_gather(mnt_v, [rb + cc1])
                    s0 = pl.ds(off + jj * (2 * L), L)
                    s1 = pl.ds(off + jj * (2 * L) + L, L)
                    canvas[s0] = jnp.maximum(canvas[s0], v0)
                    canvas[s1] = jnp.maximum(canvas[s1], v1)
                return c2

            lax.fori_loop(0, nc2, col23, 0)

        nridx = jnp.minimum(nvec[0] + lanes32, nvec[1])
        nrvec = plsc.load_gather(ctab_v, [nridx]) << 6
        return (nvec, nrvec)

    fvec0 = plsc.load_gather(fields_v, [lanes])
    ridx0 = jnp.minimum(fvec0[0] + lanes32, fvec0[1])
    rvec0 = plsc.load_gather(ctab_v, [ridx0]) << 6
    lax.fori_loop(0, N_BOXES, box_body, (fvec0, rvec0))

    copies = []
    for k in range(ROWS_PER_TILE):
        dst_off = pl.multiple_of((t + N_TILES * k) * H_IMG, 512)
        copies.append(pltpu.async_copy(
            canvas.at[pl.ds(k * H_IMG, H_IMG)],
            out_hbm.at[pl.ds(dst_off, H_IMG)], sem))
    for c in copies:
        c.wait()


@jax.jit
def _render(boxes_flat, mnt_flat):
    mesh = plsc.VectorSubcoreMesh(core_axis_name="c", subcore_axis_name="s")
    f = functools.partial(
        pl.kernel,
        mesh=mesh,
        compiler_params=pltpu.CompilerParams(needs_layout_passes=False),
        out_type=jax.ShapeDtypeStruct((W_IMG * H_IMG,), jnp.float32),
        scratch_types=[
            pltpu.VMEM((N_BOXES * 4,), jnp.float32),
            pltpu.VMEM((MNT_STRIDE * MNT_STRIDE,), jnp.float32),
            pltpu.VMEM((TDIM * TDIM,), jnp.int32),
            pltpu.VMEM(((N_BOXES + 1) * NF + L,), jnp.int32),
            pltpu.VMEM(((ROWS_PER_TILE + 1) * H_IMG,), jnp.float32),
            pltpu.SemaphoreType.DMA,
        ],
    )(_render_body)
    return f(boxes_flat, mnt_flat, jnp.asarray(_CTABLE))


def kernel(boxes, mount):
    mnt2d = mount[0, 0]
    mnt_flat = jnp.pad(mnt2d, ((0, MNT_STRIDE - MNT), (0, MNT_STRIDE - MNT)))
    mnt_flat = mnt_flat.reshape(MNT_STRIDE * MNT_STRIDE)
    boxes_flat = boxes.reshape(N_BOXES * 4)
    out = _render(boxes_flat, mnt_flat)
    return out.reshape(1, 1, W_IMG, H_IMG)


# speedup vs baseline: 1.3811x; 1.1286x over previous
import functools

import jax
import jax.numpy as jnp
import numpy as np
from jax import lax
from jax.experimental import pallas as pl
from jax.experimental.pallas import tpu as pltpu
from jax.experimental.pallas import tpu_sc as plsc

W_IMG = 512
H_IMG = 512
N_BOXES = 512
MNT = 63
MNT_STRIDE = 64
N_TILES = 32
ROWS_PER_TILE = W_IMG // N_TILES
L = 16
TDIM = 128
NF = 16

_A = np.arange(TDIM, dtype=np.int64)
_D = np.maximum(_A, 1)[:, None]
_CTABLE = np.where(_A[None, :] < _D, (_A[None, :] * MNT) // _D, MNT)
_CTABLE = _CTABLE.astype(np.int32).reshape(TDIM * TDIM)


def _render_body(boxes_hbm, mnt_hbm, ctab_hbm, out_hbm,
                 boxes_v, mnt_v, ctab_v, fields_v, canvas, sem):
    cid = lax.axis_index("c")
    sid = lax.axis_index("s")
    t = sid * 2 + cid

    c1 = pltpu.async_copy(boxes_hbm, boxes_v, sem)
    c2 = pltpu.async_copy(mnt_hbm, mnt_v, sem)
    c3 = pltpu.async_copy(ctab_hbm, ctab_v, sem)

    lanes = lax.iota(jnp.int32, L)
    lanes32 = lanes * N_TILES

    def coord_body(k, carry):
        fb = (k * L + lanes) * 4
        cx = plsc.load_gather(boxes_v, [fb])
        cy = plsc.load_gather(boxes_v, [fb + 1])
        bw = plsc.load_gather(boxes_v, [fb + 2])
        bh = plsc.load_gather(boxes_v, [fb + 3])
        x1 = ((cx - 0.5 * bw) * float(W_IMG)).astype(jnp.int32)
        y1 = ((cy - 0.5 * bh) * float(W_IMG)).astype(jnp.int32)
        x2 = ((cx + 0.5 * bw) * float(W_IMG)).astype(jnp.int32)
        y2 = ((cy + 0.5 * bh) * float(W_IMG)).astype(jnp.int32)
        w = x2 - x1
        h = y2 - y1
        wb = jnp.minimum(jnp.maximum(w, 1), TDIM - 1) << 7
        hb = jnp.minimum(jnp.maximum(h, 1), TDIM - 1) << 7
        dx0 = (t - x1) & (N_TILES - 1)
        rl0 = (x1 + dx0 - t) >> 5
        nc2 = (h + 2 * L - 1) >> 5
        off0 = (rl0 << 9) + y1
        off1 = (jnp.minimum(rl0 + 1, ROWS_PER_TILE) << 9) + y1
        off2 = (jnp.minimum(rl0 + 2, ROWS_PER_TILE) << 9) + y1
        off3 = (jnp.minimum(rl0 + 3, ROWS_PER_TILE) << 9) + y1
        fo = (k * L + lanes) * NF
        plsc.store_scatter(fields_v, [fo], wb + dx0)
        plsc.store_scatter(fields_v, [fo + 1], wb + TDIM - 1)
        plsc.store_scatter(fields_v, [fo + 2], hb)
        plsc.store_scatter(fields_v, [fo + 3], nc2)
        plsc.store_scatter(fields_v, [fo + 4], off0)
        plsc.store_scatter(fields_v, [fo + 5], off1)
        plsc.store_scatter(fields_v, [fo + 6], off2)
        plsc.store_scatter(fields_v, [fo + 7], off3)
        plsc.store_scatter(fields_v, [fo + 8], w - dx0)
        return carry

    def zero_body(k, carry):
        for u in range(4):
            canvas[pl.ds((k * 4 + u) * L, L)] = jnp.zeros((L,), jnp.float32)
        return carry

    lax.fori_loop(0, ((ROWS_PER_TILE + 1) * H_IMG) // (4 * L), zero_body, 0)

    c1.wait()
    c2.wait()
    c3.wait()

    lax.fori_loop(0, N_BOXES // L, coord_body, 0)
    plsc.store_scatter(fields_v, [N_BOXES * NF + lanes],
                       jnp.zeros((L,), jnp.int32))

    def box_body(b, carry):
        fvec, rvec = carry
        nvec = plsc.load_gather(fields_v, [(b + 1) * NF + lanes])
        hb = fvec[2]
        nc2 = fvec[3]
        off0 = fvec[4]
        off1 = fvec[5]
        off2 = fvec[6]
        off3 = fvec[7]
        nfl = fvec[8]
        rb0 = rvec[0]
        rb1 = rvec[1]

        @plsc.parallel_loop(0, nc2)
        def col01(jj):
            base = hb + jj * (2 * L)
            cc0 = ctab_v[pl.ds(base, L)]
            cc1 = ctab_v[pl.ds(base + L, L)]
            for rb, off in ((rb0, off0), (rb1, off1)):
                v0 = plsc.load_gather(mnt_v, [rb + cc0])
                v1 = plsc.load_gather(mnt_v, [rb + cc1])
                s0 = pl.ds(off + jj * (2 * L), L)
                s1 = pl.ds(off + jj * (2 * L) + L, L)
                canvas[s0] = jnp.maximum(canvas[s0], v0)
                canvas[s1] = jnp.maximum(canvas[s1], v1)

        @pl.when(nfl > 2 * N_TILES)
        def _rows23():
            rb2 = rvec[2]
            rb3 = rvec[3]

            @plsc.parallel_loop(0, nc2)
            def col23(jj):
                base = hb + jj * (2 * L)
                cc0 = ctab_v[pl.ds(base, L)]
                cc1 = ctab_v[pl.ds(base + L, L)]
                for rb, off in ((rb2, off2), (rb3, off3)):
                    v0 = plsc.load_gather(mnt_v, [rb + cc0])
                    v1 = plsc.load_gather(mnt_v, [rb + cc1])
                    s0 = pl.ds(off + jj * (2 * L), L)
                    s1 = pl.ds(off + jj * (2 * L) + L, L)
                    canvas[s0] = jnp.maximum(canvas[s0], v0)
                    canvas[s1] = jnp.maximum(canvas[s1], v1)

        nridx = jnp.minimum(nvec[0] + lanes32, nvec[1])
        nrvec = plsc.load_gather(ctab_v, [nridx]) << 6
        return (nvec, nrvec)

    fvec0 = plsc.load_gather(fields_v, [lanes])
    ridx0 = jnp.minimum(fvec0[0] + lanes32, fvec0[1])
    rvec0 = plsc.load_gather(ctab_v, [ridx0]) << 6
    lax.fori_loop(0, N_BOXES, box_body, (fvec0, rvec0))

    copies = []
    for k in range(ROWS_PER_TILE):
        dst_off = pl.multiple_of((t + N_TILES * k) * H_IMG, 512)
        copies.append(pltpu.async_copy(
            canvas.at[pl.ds(k * H_IMG, H_IMG)],
            out_hbm.at[pl.ds(dst_off, H_IMG)], sem))
    for c in copies:
        c.wait()


@jax.jit
def _render(boxes_flat, mnt_flat):
    mesh = plsc.VectorSubcoreMesh(core_axis_name="c", subcore_axis_name="s")
    f = functools.partial(
        pl.kernel,
        mesh=mesh,
        compiler_params=pltpu.CompilerParams(needs_layout_passes=False),
        out_type=jax.ShapeDtypeStruct((W_IMG * H_IMG,), jnp.float32),
        scratch_types=[
            pltpu.VMEM((N_BOXES * 4,), jnp.float32),
            pltpu.VMEM((MNT_STRIDE * MNT_STRIDE,), jnp.float32),
            pltpu.VMEM((TDIM * TDIM,), jnp.int32),
            pltpu.VMEM(((N_BOXES + 1) * NF + L,), jnp.int32),
            pltpu.VMEM(((ROWS_PER_TILE + 1) * H_IMG,), jnp.float32),
            pltpu.SemaphoreType.DMA,
        ],
    )(_render_body)
    return f(boxes_flat, mnt_flat, jnp.asarray(_CTABLE))


def kernel(boxes, mount):
    mnt2d = mount[0, 0]
    mnt_flat = jnp.pad(mnt2d, ((0, MNT_STRIDE - MNT), (0, MNT_STRIDE - MNT)))
    mnt_flat = mnt_flat.reshape(MNT_STRIDE * MNT_STRIDE)
    boxes_flat = boxes.reshape(N_BOXES * 4)
    out = _render(boxes_flat, mnt_flat)
    return out.reshape(1, 1, W_IMG, H_IMG)
